# Initial kernel scaffold; baseline (speedup 1.0000x reference)
import jax
import jax.numpy as jnp
from jax.experimental import pallas as pl


def kernel(x, centroids_acc, populations):
    raise NotImplementedError("write your pallas kernel here")


# speedup vs baseline: 43.8808x; 43.8808x over previous
import functools

import jax
import jax.numpy as jnp
from jax.experimental import pallas as pl

_NUM_CLUSTERS = 128
_PATCH = 32
_ANGLE_BINS = 8
_SPATIAL_BINS = 4
_DESC_DIM = 128
_H = 512
_W = 512
_CELL = _PATCH // _SPATIAL_BINS
_NCELL = _H // _CELL
_NPATCH = (_H // _PATCH) * (_W // _PATCH)
_SQUARINGS = 14


def _sift_cells_kernel(x_ref, out_ref):
    img = x_ref[0, 0, :, :]
    xm = jnp.concatenate([img[:, 1:], img[:, :1]], axis=1)
    xp = jnp.concatenate([img[:, -1:], img[:, :-1]], axis=1)
    ym = jnp.concatenate([img[1:, :], img[:1, :]], axis=0)
    yp = jnp.concatenate([img[-1:, :], img[:-1, :]], axis=0)
    gx = (xm - xp) * 0.5
    gy = (ym - yp) * 0.5
    mag = jnp.sqrt(gx * gx + gy * gy + 1e-12)
    ax = jnp.abs(gx)
    ay = jnp.abs(gy)
    up = gy >= 0.0
    px = gx > 0.0
    nx = gx < 0.0
    alt = ay > ax
    alb = ay < ax
    bin_hi = jnp.where(px, jnp.where(alb, 4, 5), jnp.where(alt, 6, 7))
    bin_lo = jnp.where(nx, jnp.where(alb, 0, 1), jnp.where(alt, 2, 3))
    ang = jnp.where(up, bin_hi, bin_lo).astype(jnp.int32)

    rows = jax.lax.broadcasted_iota(jnp.int32, (_H, _NCELL), 0)
    cols = jax.lax.broadcasted_iota(jnp.int32, (_H, _NCELL), 1)
    pool = (rows // _CELL == cols).astype(jnp.float32)

    for a in range(_ANGLE_BINS):
        w = jnp.where(ang == a, mag, 0.0)
        wp = jax.lax.dot_general(
            w, pool, (((1,), (0,)), ((), ())),
            preferred_element_type=jnp.float32)
        cell = jax.lax.dot_general(
            pool, wp, (((0,), (0,)), ((), ())),
            preferred_element_type=jnp.float32)
        out_ref[0, a, :, :] = cell


def _vlad_kernel(desc_ref, cent_ref, pop_ref, out_ref):
    d_raw = desc_ref[0]
    norm = jnp.sqrt(jnp.sum(d_raw * d_raw, axis=1, keepdims=True))
    d = d_raw / (norm + 1e-8)

    centroids = cent_ref[:, :] / pop_ref[:, :]

    dc = jax.lax.dot_general(
        d, centroids, (((1,), (1,)), ((), ())),
        preferred_element_type=jnp.float32)
    ones_row = jnp.full((1, _DESC_DIM), 1.0, jnp.float32)
    csq = jax.lax.dot_general(
        ones_row, centroids * centroids, (((1,), (1,)), ((), ())),
        preferred_element_type=jnp.float32)
    scores = csq - 2.0 * dc

    minv = jnp.min(scores, axis=1, keepdims=True)
    k_iota = jax.lax.broadcasted_iota(jnp.int32, (_NPATCH, _NUM_CLUSTERS), 1)
    cand = jnp.where(scores == minv, k_iota, _NUM_CLUSTERS)
    cl = jnp.min(cand, axis=1, keepdims=True)
    onehot = (k_iota == cl).astype(jnp.float32)

    desc_sums = jax.lax.dot_general(
        onehot, d, (((0,), (0,)), ((), ())),
        preferred_element_type=jnp.float32)
    ones_col = jnp.full((_NPATCH, 1), 1.0, jnp.float32)
    pops = jax.lax.dot_general(
        onehot, ones_col, (((0,), (0,)), ((), ())),
        preferred_element_type=jnp.float32)

    resid = centroids * pops - desc_sums

    A = jax.lax.dot_general(
        resid, resid, (((0,), (0,)), ((), ())),
        preferred_element_type=jnp.float32)
    B = A / jnp.sqrt(jnp.sum(A * A))
    for _ in range(_SQUARINGS):
        B2 = jax.lax.dot_general(
            B, B, (((1,), (0,)), ((), ())),
            preferred_element_type=jnp.float32)
        B = B2 / jnp.sqrt(jnp.sum(B2 * B2))
    ri = jax.lax.broadcasted_iota(jnp.int32, (_DESC_DIM, _DESC_DIM), 0)
    ci = jax.lax.broadcasted_iota(jnp.int32, (_DESC_DIM, _DESC_DIM), 1)
    eye = (ri == ci).astype(jnp.float32)
    lam = jnp.sum(A * B) / jnp.sum(B * eye)
    out_ref[0] = resid / jnp.sqrt(lam)


@jax.jit
def kernel(x, centroids_acc, populations):
    B = x.shape[0]
    cells = pl.pallas_call(
        _sift_cells_kernel,
        grid=(B,),
        in_specs=[pl.BlockSpec((1, 1, _H, _W), lambda b: (b, 0, 0, 0))],
        out_specs=pl.BlockSpec((1, _ANGLE_BINS, _NCELL, _NCELL),
                               lambda b: (b, 0, 0, 0)),
        out_shape=jax.ShapeDtypeStruct((B, _ANGLE_BINS, _NCELL, _NCELL),
                                       jnp.float32),
    )(x)

    descs_raw = (cells
                 .reshape(B, _ANGLE_BINS, 16, 4, 16, 4)
                 .transpose(0, 2, 4, 3, 5, 1)
                 .reshape(B, _NPATCH, _DESC_DIM))

    pops2 = populations.reshape(_NUM_CLUSTERS, 1)

    out = pl.pallas_call(
        _vlad_kernel,
        grid=(B,),
        in_specs=[
            pl.BlockSpec((1, _NPATCH, _DESC_DIM), lambda b: (b, 0, 0)),
            pl.BlockSpec((_NUM_CLUSTERS, _DESC_DIM), lambda b: (0, 0)),
            pl.BlockSpec((_NUM_CLUSTERS, 1), lambda b: (0, 0)),
        ],
        out_specs=pl.BlockSpec((1, _NUM_CLUSTERS, _DESC_DIM),
                               lambda b: (b, 0, 0)),
        out_shape=jax.ShapeDtypeStruct((B, _NUM_CLUSTERS, _DESC_DIM),
                                       jnp.float32),
    )(descs_raw, centroids_acc, pops2)
    return out


# speedup vs baseline: 73.0436x; 1.6646x over previous
import functools

import jax
import jax.numpy as jnp
from jax.experimental import pallas as pl

_NUM_CLUSTERS = 128
_PATCH = 32
_ANGLE_BINS = 8
_SPATIAL_BINS = 4
_DESC_DIM = 128
_H = 512
_W = 512
_CELL = _PATCH // _SPATIAL_BINS
_NCELL = _H // _CELL
_NPATCH = (_H // _PATCH) * (_W // _PATCH)
_SQUARINGS = 14


def _sift_cells_kernel(x_ref, out_ref):
    img = x_ref[0, 0, :, :]
    xm = jnp.concatenate([img[:, 1:], img[:, :1]], axis=1)
    xp = jnp.concatenate([img[:, -1:], img[:, :-1]], axis=1)
    ym = jnp.concatenate([img[1:, :], img[:1, :]], axis=0)
    yp = jnp.concatenate([img[-1:, :], img[:-1, :]], axis=0)
    gx = (xm - xp) * 0.5
    gy = (ym - yp) * 0.5
    mag = jnp.sqrt(gx * gx + gy * gy + 1e-12)
    ax = jnp.abs(gx)
    ay = jnp.abs(gy)
    up = gy >= 0.0
    px = gx > 0.0
    nx = gx < 0.0
    alt = ay > ax
    alb = ay < ax
    bin_hi = jnp.where(px, jnp.where(alb, 4, 5), jnp.where(alt, 6, 7))
    bin_lo = jnp.where(nx, jnp.where(alb, 0, 1), jnp.where(alt, 2, 3))
    ang = jnp.where(up, bin_hi, bin_lo).astype(jnp.int32)

    rows = jax.lax.broadcasted_iota(jnp.int32, (_H, _NCELL), 0)
    cols = jax.lax.broadcasted_iota(jnp.int32, (_H, _NCELL), 1)
    pool = (rows // _CELL == cols).astype(jnp.float32)

    xc_rows = jax.lax.broadcasted_iota(jnp.int32, (_NCELL, _H), 0)
    m_cols = jax.lax.broadcasted_iota(jnp.int32, (_NCELL, _H), 1)
    acc = jnp.zeros((_NCELL, _H), jnp.float32)
    for a in range(_ANGLE_BINS):
        w = jnp.where(ang == a, mag, 0.0)
        wp = jax.lax.dot_general(
            w, pool, (((1,), (0,)), ((), ())),
            preferred_element_type=jnp.float32)
        cell = jax.lax.dot_general(
            pool, wp, (((0,), (0,)), ((), ())),
            preferred_element_type=jnp.float32)
        spread = (m_cols == xc_rows * _ANGLE_BINS + a).astype(jnp.float32)
        acc = acc + jax.lax.dot_general(
            cell, spread, (((1,), (0,)), ((), ())),
            preferred_element_type=jnp.float32)
    out_ref[0] = acc


def _vlad_kernel(desc_ref, cent_ref, pop_ref, out_ref):
    nb = desc_ref.shape[0]
    centroids = cent_ref[:, :] / pop_ref[:, :]

    d_all = desc_ref[...].reshape(nb * _NPATCH, _DESC_DIM)
    norm = jnp.sqrt(jnp.sum(d_all * d_all, axis=1, keepdims=True))
    d_all = d_all / (norm + 1e-8)

    dc = jax.lax.dot_general(
        d_all, centroids, (((1,), (1,)), ((), ())),
        preferred_element_type=jnp.float32)
    ones_row = jnp.full((1, _DESC_DIM), 1.0, jnp.float32)
    csq = jax.lax.dot_general(
        ones_row, centroids * centroids, (((1,), (1,)), ((), ())),
        preferred_element_type=jnp.float32)
    scores = csq - 2.0 * dc

    minv = jnp.min(scores, axis=1, keepdims=True)
    k_iota = jax.lax.broadcasted_iota(
        jnp.int32, (nb * _NPATCH, _NUM_CLUSTERS), 1)
    cand = jnp.where(scores == minv, k_iota, _NUM_CLUSTERS)
    cl = jnp.min(cand, axis=1, keepdims=True)
    onehot = (k_iota == cl).astype(jnp.float32)

    ones_col = jnp.full((_NPATCH, 1), 1.0, jnp.float32)
    resids = []
    mats = []
    for b in range(nb):
        oh = onehot[b * _NPATCH:(b + 1) * _NPATCH]
        db = d_all[b * _NPATCH:(b + 1) * _NPATCH]
        desc_sums = jax.lax.dot_general(
            oh, db, (((0,), (0,)), ((), ())),
            preferred_element_type=jnp.float32)
        pops = jax.lax.dot_general(
            oh, ones_col, (((0,), (0,)), ((), ())),
            preferred_element_type=jnp.float32)
        resid = centroids * pops - desc_sums
        resids.append(resid)
        A = jax.lax.dot_general(
            resid, resid, (((0,), (0,)), ((), ())),
            preferred_element_type=jnp.float32)
        mats.append((A, A / jnp.sqrt(jnp.sum(A * A))))

    for _ in range(_SQUARINGS):
        for b in range(nb):
            A, B = mats[b]
            B2 = jax.lax.dot_general(
                B, B, (((1,), (0,)), ((), ())),
                preferred_element_type=jnp.float32)
            mats[b] = (A, B2 / jnp.sqrt(jnp.sum(B2 * B2)))
    ri = jax.lax.broadcasted_iota(jnp.int32, (_DESC_DIM, _DESC_DIM), 0)
    ci = jax.lax.broadcasted_iota(jnp.int32, (_DESC_DIM, _DESC_DIM), 1)
    eye = (ri == ci).astype(jnp.float32)
    for b in range(nb):
        A, B = mats[b]
        lam = jnp.sum(A * B) / jnp.sum(B * eye)
        out_ref[b] = resids[b] / jnp.sqrt(lam)


@jax.jit
def kernel(x, centroids_acc, populations):
    B = x.shape[0]
    cells = pl.pallas_call(
        _sift_cells_kernel,
        grid=(B,),
        in_specs=[pl.BlockSpec((1, 1, _H, _W), lambda b: (b, 0, 0, 0))],
        out_specs=pl.BlockSpec((1, _NCELL, _H),
                               lambda b: (b, 0, 0)),
        out_shape=jax.ShapeDtypeStruct((B, _NCELL, _H),
                                       jnp.float32),
    )(x)

    descs_raw = (cells
                 .reshape(B, 16, 4, 16, 32)
                 .transpose(0, 1, 3, 2, 4)
                 .reshape(B, _NPATCH, _DESC_DIM))

    pops2 = populations.reshape(_NUM_CLUSTERS, 1)

    out = pl.pallas_call(
        _vlad_kernel,
        in_specs=[
            pl.BlockSpec((B, _NPATCH, _DESC_DIM), lambda: (0, 0, 0)),
            pl.BlockSpec((_NUM_CLUSTERS, _DESC_DIM), lambda: (0, 0)),
            pl.BlockSpec((_NUM_CLUSTERS, 1), lambda: (0, 0)),
        ],
        out_specs=pl.BlockSpec((B, _NUM_CLUSTERS, _DESC_DIM),
                               lambda: (0, 0, 0)),
        out_shape=jax.ShapeDtypeStruct((B, _NUM_CLUSTERS, _DESC_DIM),
                                       jnp.float32),
    )(descs_raw, centroids_acc, pops2)
    return out
